# baseline (speedup 1.0000x reference)
import jax
import jax.numpy as jnp
from jax import lax
from jax.experimental import pallas as pl
from jax.experimental.pallas import tpu as pltpu
from jax.experimental.pallas import tpu_sc as plsc

B = 16384
K = 32
NR = 2
NC, NS, L = 2, 16, 16
NW = NC * NS
RPW = B // NW
CH = 16
NCHUNK = RPW // CH
WU = 4 * NR * K
WI = 4 * K

_LN2 = 0.6931471805599453
_SQRT2 = 1.4142135623730951


def _vlog(a):
    ab = lax.bitcast_convert_type(a, jnp.int32)
    e = lax.shift_right_logical(ab, 23) - 127
    m = lax.bitcast_convert_type(
        jnp.bitwise_or(jnp.bitwise_and(ab, 0x007FFFFF), 0x3F800000),
        jnp.float32)
    big = m > _SQRT2
    m = jnp.where(big, m * 0.5, m)
    ef = (e + jnp.where(big, 1, 0)).astype(jnp.float32)
    t = (m - 1.0) / (m + 1.0)
    t2 = t * t
    p = 2.0 + t2 * (2.0 / 3.0 + t2 * (2.0 / 5.0 + t2 * (2.0 / 7.0 + t2 * (2.0 / 9.0))))
    return ef * _LN2 + t * p


def _rule_neg_log(s):
    sig = 1.0 / (1.0 + jnp.exp(-s))
    return _vlog((1.0 - sig) + 1e-40)


def _body(users_r, items_r, gu_tab, gi_tab, xui_o, gu_o, gi_o,
          idx_u, idx_i, grp_u, sub_u, grp_i, sub_i,
          bufU0, bufU1, bufI0, bufI1,
          stgU0, stgU1, stgI0, stgI1, xui_v,
          semU0, semU1, semI0, semI1):
    wid = lax.axis_index("s") * NC + lax.axis_index("c")
    base = wid * RPW
    iota = lax.iota(jnp.int32, L)

    pltpu.sync_copy(users_r.at[pl.ds(base, RPW)], idx_u)
    pltpu.sync_copy(items_r.at[pl.ds(base, RPW)], idx_i)

    def split(g, _):
        u = idx_u[pl.ds(g * L, L)]
        grp_u[pl.ds(g * L, L)] = lax.shift_right_logical(u, 2)
        sub_u[pl.ds(g * L, L)] = jnp.bitwise_and(u, 3)
        i = idx_i[pl.ds(g * L, L)]
        grp_i[pl.ds(g * L, L)] = lax.shift_right_logical(i, 2)
        sub_i[pl.ds(g * L, L)] = jnp.bitwise_and(i, 3)
        return _

    lax.fori_loop(0, NCHUNK, split, 0)

    def fire(c, bufU, bufI, semU, semI):
        pltpu.async_copy(gu_tab.at[grp_u.at[pl.ds(c * CH, CH)]], bufU, semU)
        pltpu.async_copy(gi_tab.at[grp_i.at[pl.ds(c * CH, CH)]], bufI, semI)

    def wait(c, bufU, bufI, semU, semI):
        pltpu.make_async_copy(gu_tab.at[grp_u.at[pl.ds(c * CH, CH)]], bufU, semU).wait()
        pltpu.make_async_copy(gi_tab.at[grp_i.at[pl.ds(c * CH, CH)]], bufI, semI).wait()

    def process(c, bufU, bufI, stgU, stgI):
        sv_u = sub_u[pl.ds(c * CH, CH)] * (NR * K)
        sv_i = sub_i[pl.ds(c * CH, CH)] * K
        for k in range(NR * K):
            v = plsc.load_gather(bufU, [iota, sv_u + k])
            plsc.store_scatter(stgU, [iota * (NR * K) + k], v)
        for k in range(K):
            v = plsc.load_gather(bufI, [iota, sv_i + k])
            plsc.store_scatter(stgI, [iota * K + k], v)
        a0 = jnp.zeros((L,), jnp.float32)
        a1 = jnp.zeros((L,), jnp.float32)
        for r in range(CH):
            ia = stgI[pl.ds(r * K, L)]
            ib = stgI[pl.ds(r * K + L, L)]
            u0a = stgU[pl.ds(r * NR * K, L)]
            u0b = stgU[pl.ds(r * NR * K + L, L)]
            u1a = stgU[pl.ds(r * NR * K + 2 * L, L)]
            u1b = stgU[pl.ds(r * NR * K + 3 * L, L)]
            s0 = jnp.sum(u0a * ia + u0b * ib)
            s1 = jnp.sum(u1a * ia + u1b * ib)
            sel = iota == r
            a0 = jnp.where(sel, s0, a0)
            a1 = jnp.where(sel, s1, a1)
        log_sum = _rule_neg_log(a0) + _rule_neg_log(a1)
        xui_v[pl.ds(c * CH, L)] = 1.0 - (-1.0 / (-1.0 + log_sum))
        row0 = base + c * CH
        pltpu.sync_copy(stgU, gu_o.at[pl.ds(row0 * NR * K, CH * NR * K)])
        pltpu.sync_copy(stgI, gi_o.at[pl.ds(row0 * K, CH * K)])

    fire(0, bufU0, bufI0, semU0, semI0)

    def pair(t, carry):
        c0 = 2 * t
        c1 = 2 * t + 1
        fire(c1, bufU1, bufI1, semU1, semI1)
        wait(c0, bufU0, bufI0, semU0, semI0)
        process(c0, bufU0, bufI0, stgU0, stgI0)

        @pl.when(t < (NCHUNK // 2 - 1))
        def _():
            fire(c0 + 2, bufU0, bufI0, semU0, semI0)

        wait(c1, bufU1, bufI1, semU1, semI1)
        process(c1, bufU1, bufI1, stgU1, stgI1)
        return carry

    lax.fori_loop(0, NCHUNK // 2, pair, 0)
    pltpu.sync_copy(xui_v, xui_o.at[pl.ds(base, RPW)])


def kernel(users, items, Gu, Gi):
    users = users.astype(jnp.int32)
    items = items.astype(jnp.int32)
    gu4 = Gu.reshape(-1, WU)
    gi4 = Gi.reshape(-1, WI)
    run = pl.kernel(
        _body,
        out_type=(
            jax.ShapeDtypeStruct((B,), jnp.float32),
            jax.ShapeDtypeStruct((B * NR * K,), jnp.float32),
            jax.ShapeDtypeStruct((B * K,), jnp.float32),
        ),
        mesh=plsc.VectorSubcoreMesh(core_axis_name="c", subcore_axis_name="s"),
        scratch_types=(
            pltpu.VMEM((RPW,), jnp.int32),
            pltpu.VMEM((RPW,), jnp.int32),
            pltpu.VMEM((RPW,), jnp.int32),
            pltpu.VMEM((RPW,), jnp.int32),
            pltpu.VMEM((RPW,), jnp.int32),
            pltpu.VMEM((RPW,), jnp.int32),
            pltpu.VMEM((CH, WU), jnp.float32),
            pltpu.VMEM((CH, WU), jnp.float32),
            pltpu.VMEM((CH, WI), jnp.float32),
            pltpu.VMEM((CH, WI), jnp.float32),
            pltpu.VMEM((CH * NR * K,), jnp.float32),
            pltpu.VMEM((CH * NR * K,), jnp.float32),
            pltpu.VMEM((CH * K,), jnp.float32),
            pltpu.VMEM((CH * K,), jnp.float32),
            pltpu.VMEM((RPW,), jnp.float32),
            pltpu.SemaphoreType.DMA,
            pltpu.SemaphoreType.DMA,
            pltpu.SemaphoreType.DMA,
            pltpu.SemaphoreType.DMA,
        ),
        compiler_params=pltpu.CompilerParams(
            needs_layout_passes=False, use_tc_tiling_on_sc=True),
    )
    xui, gu_flat, gamma_flat = run(users, items, gu4, gi4)
    return xui, gu_flat.reshape(B, NR, K), gamma_flat.reshape(B, K)


# speedup vs baseline: 1.1402x; 1.1402x over previous
import jax
import jax.numpy as jnp
from jax import lax
from jax.experimental import pallas as pl
from jax.experimental.pallas import tpu as pltpu
from jax.experimental.pallas import tpu_sc as plsc

B = 16384
K = 32
NR = 2
W = 128
NC, NS, L = 2, 16, 16
NW = NC * NS
RPW = B // NW
CHK = 128

_LN2 = 0.6931471805599453
_SQRT2 = 1.4142135623730951


def _vlog(a):
    ab = lax.bitcast_convert_type(a, jnp.int32)
    e = lax.shift_right_logical(ab, 23) - 127
    m = lax.bitcast_convert_type(
        jnp.bitwise_or(jnp.bitwise_and(ab, 0x007FFFFF), 0x3F800000),
        jnp.float32)
    big = m > _SQRT2
    m = jnp.where(big, m * 0.5, m)
    ef = (e + jnp.where(big, 1, 0)).astype(jnp.float32)
    t = (m - 1.0) / (m + 1.0)
    t2 = t * t
    p = 2.0 + t2 * (2.0 / 3.0 + t2 * (2.0 / 5.0 + t2 * (2.0 / 7.0 + t2 * (2.0 / 9.0))))
    return ef * _LN2 + t * p


def _rule_neg_log(s):
    sig = 1.0 / (1.0 + jnp.exp(-s))
    return _vlog((1.0 - sig) + 1e-40)


def _body(users_r, items_r, gu_tab, gi_tab, xui_o, gu_o, gi_o,
          idx_u, idx_i, gu_v, gi_v, xui_v, sem_g, sem_o):
    wid = lax.axis_index("s") * NC + lax.axis_index("c")
    base = wid * RPW

    pltpu.sync_copy(users_r.at[pl.ds(base, RPW)], idx_u)
    pltpu.sync_copy(items_r.at[pl.ds(base, RPW)], idx_i)

    iota = lax.iota(jnp.int32, L)

    def chunk(c, carry0):
        cu = pltpu.async_copy(gu_tab.at[idx_u.at[pl.ds(c * CHK, CHK)]], gu_v, sem_g)
        ci = pltpu.async_copy(gi_tab.at[idx_i.at[pl.ds(c * CHK, CHK)]], gi_v, sem_g)
        cu.wait()
        ci.wait()

        def group(g, carry):
            def rowfn(r, accs):
                a0, a1 = accs
                b = g * L + r
                ia = gi_v[b, pl.ds(0, L)]
                ib = gi_v[b, pl.ds(L, L)]
                u0a = gu_v[b, pl.ds(0, L)]
                u0b = gu_v[b, pl.ds(L, L)]
                u1a = gu_v[b, pl.ds(2 * L, L)]
                u1b = gu_v[b, pl.ds(3 * L, L)]
                s0 = jnp.sum(u0a * ia + u0b * ib)
                s1 = jnp.sum(u1a * ia + u1b * ib)
                sel = iota == r
                return (jnp.where(sel, s0, a0), jnp.where(sel, s1, a1))

            z = jnp.zeros((L,), jnp.float32)
            a0, a1 = lax.fori_loop(0, L, rowfn, (z, z))
            log_sum = _rule_neg_log(a0) + _rule_neg_log(a1)
            xui_v[pl.ds(c * CHK + g * L, L)] = 1.0 - (-1.0 / (-1.0 + log_sum))
            return carry

        lax.fori_loop(0, CHK // L, group, 0)
        pltpu.sync_copy(gu_v, gu_o.at[pl.ds(base + c * CHK, CHK)])
        pltpu.sync_copy(gi_v, gi_o.at[pl.ds(base + c * CHK, CHK)])
        return carry0

    lax.fori_loop(0, RPW // CHK, chunk, 0)
    pltpu.sync_copy(xui_v, xui_o.at[pl.ds(base, RPW)])


def kernel(users, items, Gu, Gi):
    users = users.astype(jnp.int32)
    items = items.astype(jnp.int32)
    gu_p = jnp.pad(Gu, ((0, 0), (0, W - NR * K)))
    gi_p = jnp.pad(Gi, ((0, 0), (0, W - K)))
    run = pl.kernel(
        _body,
        out_type=(
            jax.ShapeDtypeStruct((B,), jnp.float32),
            jax.ShapeDtypeStruct((B, W), jnp.float32),
            jax.ShapeDtypeStruct((B, W), jnp.float32),
        ),
        mesh=plsc.VectorSubcoreMesh(core_axis_name="c", subcore_axis_name="s"),
        scratch_types=(
            pltpu.VMEM((RPW,), jnp.int32),
            pltpu.VMEM((RPW,), jnp.int32),
            pltpu.VMEM((CHK, W), jnp.float32),
            pltpu.VMEM((CHK, W), jnp.float32),
            pltpu.VMEM((RPW,), jnp.float32),
            pltpu.SemaphoreType.DMA,
            pltpu.SemaphoreType.DMA,
        ),
        compiler_params=pltpu.CompilerParams(
            needs_layout_passes=False, use_tc_tiling_on_sc=True),
    )
    xui, gu_pad, gi_pad = run(users, items, gu_p, gi_p)
    return xui, gu_pad[:, :NR * K].reshape(B, NR, K), gi_pad[:, :K]
